# baseline (speedup 1.0000x reference)
import functools

import jax
import jax.numpy as jnp
from jax import lax
from jax.experimental import pallas as pl
from jax.experimental.pallas import tpu as pltpu
from jax.experimental.pallas import tpu_sc as plsc

B = 1024
L = 50
OUT_V = 1099
LANES = 16
NC = 2
NS = 16

LT = 7
BT = 8
VQ = 35
VT_TOTAL = 138


@functools.partial(
    pl.kernel,
    mesh=plsc.VectorSubcoreMesh(core_axis_name="c", subcore_axis_name="s"),
    out_type=jax.ShapeDtypeStruct((VT_TOTAL, BT, 8, 128), jnp.float32),
    scratch_types=[
        pltpu.VMEM((LT, 1, 8, 128), jnp.int32),
        pltpu.VMEM((VQ, 1, 8, 128), jnp.float32),
    ],
    compiler_params=pltpu.CompilerParams(
        needs_layout_passes=False,
        disable_bounds_checks=True,
        skip_device_barrier=True,
    ),
)
def _bag_of_words(in_hbm, out_hbm, tok_v, acc_v):
    wid = lax.axis_index("s") * NC + lax.axis_index("c")
    bt = wid >> 2
    q = wid & 3
    vt0 = q * VQ - jnp.maximum(q - 2, 0)
    nvt8 = (VQ - 1) * 8 + jnp.where(q < 2, 8, 0)
    vbase = vt0 * 8

    pltpu.sync_copy(in_hbm.at[:, pl.ds(bt, 1)], tok_v)

    lane = lax.iota(jnp.int32, LANES)
    zf = jnp.zeros((LANES,), jnp.float32)
    ones = jnp.ones((LANES,), jnp.float32)
    zidx = jnp.zeros((LANES,), jnp.int32)

    def zero_vt(vtl, carry):
        for r in range(8):
            for k in range(8):
                acc_v[vtl, 0, r, pl.ds(k * LANES, LANES)] = zf
        return carry

    lax.fori_loop(0, VQ, zero_vt, 0)

    def count_chunk(cc, carry):
        bc = lane + cc * LANES
        for l in range(L):
            tok = tok_v[l // 8, 0, l % 8, pl.ds(cc * LANES, LANES)]
            v = tok - 1
            m = (v >= vbase) & (v < vbase + nvt8)
            lv = jnp.minimum(jnp.maximum(v - vbase, 0), VQ * 8 - 1)
            plsc.addupdate_scatter(
                acc_v, [lv >> 3, zidx, lv & 7, bc], ones, mask=m
            )
        return carry

    lax.fori_loop(0, BT, count_chunk, 0)

    @pl.when(q < 2)
    def _():
        pltpu.sync_copy(
            acc_v.at[pl.ds(0, VQ)], out_hbm.at[pl.ds(vt0, VQ), pl.ds(bt, 1)]
        )

    @pl.when(q >= 2)
    def _():
        pltpu.sync_copy(
            acc_v.at[pl.ds(0, VQ - 1)],
            out_hbm.at[pl.ds(vt0, VQ - 1), pl.ds(bt, 1)],
        )


def kernel(inputs):
    padt = jnp.pad(inputs.T, ((0, 6), (0, 0)))
    in4 = padt.reshape(LT, 8, BT, 128).transpose(0, 2, 1, 3)
    out4 = _bag_of_words(in4)
    out = out4.transpose(0, 2, 1, 3).reshape(VT_TOTAL * 8, B)
    return out[:OUT_V].T


# speedup vs baseline: 1.1878x; 1.1878x over previous
import functools

import jax
import jax.numpy as jnp
from jax import lax
from jax.experimental import pallas as pl
from jax.experimental.pallas import tpu as pltpu
from jax.experimental.pallas import tpu_sc as plsc

B = 1024
L = 50
OUT_V = 1099
LANES = 16
NC = 2
NS = 16

LT = 7
BT = 8
VQ = 35
VT_TOTAL = 138


@functools.partial(
    pl.kernel,
    mesh=plsc.VectorSubcoreMesh(core_axis_name="c", subcore_axis_name="s"),
    out_type=jax.ShapeDtypeStruct((VT_TOTAL, BT, 8, 128), jnp.float32),
    scratch_types=[
        pltpu.VMEM((LT, 1, 8, 128), jnp.int32),
        pltpu.VMEM((VQ, 8, 128), jnp.float32),
    ],
    compiler_params=pltpu.CompilerParams(
        needs_layout_passes=False,
        disable_bounds_checks=True,
        skip_device_barrier=True,
    ),
)
def _bag_of_words(in_hbm, out_hbm, tok_v, acc_v):
    wid = lax.axis_index("s") * NC + lax.axis_index("c")
    bt = wid >> 2
    q = wid & 3
    vt0 = q * VQ - jnp.maximum(q - 2, 0)
    nvt8 = (VQ - 1) * 8 + jnp.where(q < 2, 8, 0)
    vbase = vt0 * 8

    pltpu.sync_copy(in_hbm.at[:, pl.ds(bt, 1)], tok_v)

    lane = lax.iota(jnp.int32, LANES)
    zf = jnp.zeros((LANES,), jnp.float32)
    ones = jnp.ones((LANES,), jnp.float32)
    zidx = jnp.zeros((LANES,), jnp.int32)

    @plsc.parallel_loop(0, VQ, 1, unroll=2)
    def _zero(vtl):
        for r in range(8):
            for k in range(8):
                acc_v[vtl, r, pl.ds(k * LANES, LANES)] = zf

    @plsc.parallel_loop(0, BT, 1, unroll=2)
    def _count(cc):
        bc = lane + cc * LANES
        for l in range(L):
            tok = tok_v[l // 8, 0, l % 8, pl.ds(cc * LANES, LANES)]
            d = tok - (vbase + 1)
            m = d.astype(jnp.uint32) < nvt8.astype(jnp.uint32)
            lv = jnp.minimum(jnp.maximum(d, 0), VQ * 8 - 1)
            plsc.addupdate_scatter(acc_v, [lv >> 3, lv & 7, bc], ones, mask=m)

    @pl.when(q < 2)
    def _():
        pltpu.sync_copy(
            acc_v.at[pl.ds(0, VQ)], out_hbm.at[pl.ds(vt0, VQ), bt]
        )

    @pl.when(q >= 2)
    def _():
        pltpu.sync_copy(
            acc_v.at[pl.ds(0, VQ - 1)],
            out_hbm.at[pl.ds(vt0, VQ - 1), bt],
        )


def kernel(inputs):
    padt = jnp.pad(inputs.T, ((0, 6), (0, 0)))
    in4 = padt.reshape(LT, 8, BT, 128).transpose(0, 2, 1, 3)
    out4 = _bag_of_words(in4)
    out = out4.transpose(0, 2, 1, 3).reshape(VT_TOTAL * 8, B)
    return out.T[:, :OUT_V]


# speedup vs baseline: 1.2758x; 1.0741x over previous
import functools

import jax
import jax.numpy as jnp
from jax import lax
from jax.experimental import pallas as pl
from jax.experimental.pallas import tpu as pltpu
from jax.experimental.pallas import tpu_sc as plsc

B = 1024
L = 50
OUT_V = 1099
LANES = 16
NC = 2
NS = 16

LT = 7
BT = 8
VQ = 35
VT_TOTAL = 138


@functools.partial(
    pl.kernel,
    mesh=plsc.VectorSubcoreMesh(core_axis_name="c", subcore_axis_name="s"),
    out_type=jax.ShapeDtypeStruct((VT_TOTAL, BT, 8, 128), jnp.float32),
    scratch_types=[
        pltpu.VMEM((LT, 1, 8, 128), jnp.int32),
        pltpu.VMEM((VQ, 8, 128), jnp.float32),
        pltpu.SemaphoreType.DMA,
    ],
    compiler_params=pltpu.CompilerParams(
        needs_layout_passes=False,
        disable_bounds_checks=True,
        skip_device_barrier=True,
    ),
)
def _bag_of_words(in_hbm, out_hbm, tok_v, acc_v, sem):
    wid = lax.axis_index("s") * NC + lax.axis_index("c")
    bt = wid >> 2
    q = wid & 3
    vt0 = q * VQ - jnp.maximum(q - 2, 0)
    nvt8 = (VQ - 1) * 8 + jnp.where(q < 2, 8, 0)
    vbase = vt0 * 8

    in_copy = pltpu.async_copy(in_hbm.at[:, pl.ds(bt, 1)], tok_v, sem)

    lane = lax.iota(jnp.int32, LANES)
    zf = jnp.zeros((LANES,), jnp.float32)
    ones = jnp.ones((LANES,), jnp.float32)

    @plsc.parallel_loop(0, VQ, 1, unroll=4)
    def _zero(vtl):
        for r in range(8):
            for k in range(8):
                acc_v[vtl, r, pl.ds(k * LANES, LANES)] = zf

    in_copy.wait()

    @plsc.parallel_loop(0, BT, 1, unroll=4)
    def _count(cc):
        bc = lane + cc * LANES
        for l in range(L):
            tok = tok_v[l // 8, 0, l % 8, pl.ds(cc * LANES, LANES)]
            d = tok - (vbase + 1)
            m = d.astype(jnp.uint32) < nvt8.astype(jnp.uint32)
            lv = jnp.minimum(jnp.maximum(d, 0), VQ * 8 - 1)
            plsc.addupdate_scatter(acc_v, [lv >> 3, lv & 7, bc], ones, mask=m)

    @pl.when(q < 2)
    def _():
        pltpu.sync_copy(
            acc_v.at[pl.ds(0, VQ)], out_hbm.at[pl.ds(vt0, VQ), bt]
        )

    @pl.when(q >= 2)
    def _():
        pltpu.sync_copy(
            acc_v.at[pl.ds(0, VQ - 1)],
            out_hbm.at[pl.ds(vt0, VQ - 1), bt],
        )


def kernel(inputs):
    padt = jnp.pad(inputs.T, ((0, 6), (0, 0)))
    in4 = padt.reshape(LT, 8, BT, 128).transpose(0, 2, 1, 3)
    out4 = _bag_of_words(in4)
    out = out4.transpose(0, 2, 1, 3).reshape(VT_TOTAL * 8, B)
    return out.T[:, :OUT_V]
